# baseline (speedup 1.0000x reference)
import jax
import jax.numpy as jnp
from jax.experimental import pallas as pl

N = 21
E = 168


def _fused_body(x0_ref, adj_ref, w1_ref, b1_ref, w2_ref, b2_ref,
                fw1_ref, fb1_ref, fw2_ref, fb2_ref, fw3_ref, fb3_ref,
                out_ref):
    f32 = jnp.float32
    src = adj_ref[0:1, :]
    dst = adj_ref[1:2, :]
    node_iota = jax.lax.broadcasted_iota(jnp.int32, (N, E), 0)
    S = (src == node_iota).astype(f32)
    D = (dst == node_iota).astype(f32)
    C = jax.lax.dot_general(D, S, (((1,), (1,)), ((), ())),
                            preferred_element_type=f32)
    deg = jnp.sum(C, axis=1, keepdims=True) + 1.0
    dinv = jax.lax.rsqrt(deg)
    eye = (jax.lax.broadcasted_iota(jnp.int32, (N, N), 0)
           == jax.lax.broadcasted_iota(jnp.int32, (N, N), 1)).astype(f32)
    a_hat = C * dinv * dinv.reshape(1, N) + eye * (dinv * dinv)

    xw1 = jnp.dot(x0_ref[:], w1_ref[:], preferred_element_type=f32)
    x1 = jnp.dot(a_hat, xw1, preferred_element_type=f32) + b1_ref[:]
    xw2 = jnp.dot(x1, w2_ref[:], preferred_element_type=f32)
    x2 = jnp.dot(a_hat, xw2, preferred_element_type=f32) + b2_ref[:]

    flat = jnp.concatenate([x2[n:n + 1, :] for n in range(N)], axis=1)
    h1 = jnp.maximum(
        jnp.dot(flat.astype(jnp.bfloat16), fw1_ref[:],
                preferred_element_type=f32) + fb1_ref[:], 0.0)
    h2 = jnp.maximum(
        jnp.dot(h1.astype(jnp.bfloat16), fw2_ref[:],
                preferred_element_type=f32) + fb2_ref[:], 0.0)
    h3 = jnp.maximum(
        jnp.dot(h2.astype(jnp.bfloat16), fw3_ref[:],
                preferred_element_type=f32) + fb3_ref[:], 0.0)
    out_ref[:] = h3


def kernel(state, adj, W1, b1, W2, b2, fW1, fb1, fW2, fb2, fW3, fb3):
    x0 = state[0]
    out = pl.pallas_call(
        _fused_body,
        out_shape=jax.ShapeDtypeStruct((1, 18), jnp.float32),
    )(x0, adj, W1, b1.reshape(1, N), W2, b2.reshape(1, N),
      fW1.astype(jnp.bfloat16), fb1.reshape(1, 512),
      fW2.astype(jnp.bfloat16), fb2.reshape(1, 512),
      fW3.astype(jnp.bfloat16), fb3.reshape(1, 18))
    return out.reshape(18)


# speedup vs baseline: 1.4638x; 1.4638x over previous
import jax
import jax.numpy as jnp
from jax.experimental import pallas as pl

N = 21
E = 168


def _fused_body(x0_ref, adj_ref, w1_ref, b1_ref, w2_ref, b2_ref,
                fw1_ref, fb1_ref, fw2_ref, fb2_ref, fw3_ref, fb3_ref,
                out_ref):
    f32 = jnp.float32
    src = adj_ref[0:1, :]
    dst = adj_ref[1:2, :]
    node_iota = jax.lax.broadcasted_iota(jnp.int32, (N, E), 0)
    S = (src == node_iota).astype(f32)
    D = (dst == node_iota).astype(f32)
    C = jax.lax.dot_general(D, S, (((1,), (1,)), ((), ())),
                            preferred_element_type=f32)
    deg = jnp.sum(C, axis=1, keepdims=True) + 1.0
    dinv = jax.lax.rsqrt(deg)
    eye = (jax.lax.broadcasted_iota(jnp.int32, (N, N), 0)
           == jax.lax.broadcasted_iota(jnp.int32, (N, N), 1)).astype(f32)
    a_hat = C * dinv * dinv.reshape(1, N) + eye * (dinv * dinv)

    xw1 = jnp.dot(x0_ref[:], w1_ref[:], preferred_element_type=f32)
    x1 = jnp.dot(a_hat, xw1, preferred_element_type=f32) + b1_ref[:]
    xw2 = jnp.dot(x1, w2_ref[:], preferred_element_type=f32)
    x2 = jnp.dot(a_hat, xw2, preferred_element_type=f32) + b2_ref[:]

    flat = jnp.concatenate([x2[n:n + 1, :] for n in range(N)], axis=1)
    h1 = jnp.maximum(jnp.dot(flat, fw1_ref[:], preferred_element_type=f32)
                     + fb1_ref[:], 0.0)
    h2 = jnp.maximum(jnp.dot(h1, fw2_ref[:], preferred_element_type=f32)
                     + fb2_ref[:], 0.0)
    h3 = jnp.maximum(jnp.dot(h2, fw3_ref[:], preferred_element_type=f32)
                     + fb3_ref[:], 0.0)
    out_ref[:] = h3


def kernel(state, adj, W1, b1, W2, b2, fW1, fb1, fW2, fb2, fW3, fb3):
    x0 = state[0]
    out = pl.pallas_call(
        _fused_body,
        out_shape=jax.ShapeDtypeStruct((1, 18), jnp.float32),
    )(x0, adj, W1, b1.reshape(1, N), W2, b2.reshape(1, N),
      fW1, fb1.reshape(1, 512), fW2, fb2.reshape(1, 512),
      fW3, fb3.reshape(1, 18))
    return out.reshape(18)
